# baseline (speedup 1.0000x reference)
import functools

import jax
import jax.numpy as jnp
from jax import lax
from jax.experimental import pallas as pl
from jax.experimental.pallas import tpu as pltpu
from jax.experimental.pallas import tpu_sc as plsc

_LANES = 16


@functools.lru_cache(maxsize=None)
def _build(num_ent, num_rel, dim, batch):
    info = plsc.get_sparse_core_info()
    nc, ns = info.num_cores, info.num_subcores
    nw = nc * ns
    assert batch % (8 * nw) == 0 and dim % _LANES == 0
    bpw = batch // nw
    nchunk = dim // _LANES

    mesh = plsc.VectorSubcoreMesh(core_axis_name="c", subcore_axis_name="s")

    @functools.partial(
        pl.kernel,
        mesh=mesh,
        out_type=jax.ShapeDtypeStruct((batch, dim), jnp.float32),
        compiler_params=pltpu.CompilerParams(use_tc_tiling_on_sc=False),
        scratch_types=[
            pltpu.VMEM((bpw,), jnp.int32),
            pltpu.VMEM((bpw,), jnp.int32),
            pltpu.VMEM((bpw,), jnp.int32),
            pltpu.VMEM((bpw, dim), jnp.float32),
            pltpu.VMEM((bpw, dim), jnp.float32),
            pltpu.VMEM((bpw, dim), jnp.float32),
            pltpu.SemaphoreType.DMA,
            pltpu.SemaphoreType.DMA,
            pltpu.SemaphoreType.DMA,
        ],
    )
    def k(ent_hbm, rel_hbm, h_hbm, t_hbm, r_hbm, out_hbm,
          hidx, tidx, ridx, hrow, trow, rrow, sem_h, sem_t, sem_r):
        wid = lax.axis_index("s") * nc + lax.axis_index("c")
        base = wid * bpw
        pltpu.sync_copy(h_hbm.at[pl.ds(base, bpw)], hidx)
        pltpu.sync_copy(t_hbm.at[pl.ds(base, bpw)], tidx)
        pltpu.sync_copy(r_hbm.at[pl.ds(base, bpw)], ridx)
        ch = pltpu.async_copy(ent_hbm.at[hidx], hrow, sem_h)
        ct = pltpu.async_copy(ent_hbm.at[tidx], trow, sem_t)
        cr = pltpu.async_copy(rel_hbm.at[ridx], rrow, sem_r)
        ch.wait()
        ct.wait()
        cr.wait()

        def row_body(i, carry):
            for c in range(nchunk):
                sl = (i, pl.ds(c * _LANES, _LANES))
                hrow[sl] = hrow[sl] + rrow[sl] - trow[sl]
            return carry

        lax.fori_loop(0, bpw, row_body, 0)
        pltpu.sync_copy(hrow, out_hbm.at[pl.ds(base, bpw)])

    return k


def kernel(ent_table, rel_table, h_list, t_list, r_list):
    num_ent, dim = ent_table.shape
    num_rel = rel_table.shape[0]
    batch = h_list.shape[0]
    k = _build(num_ent, num_rel, dim, batch)
    return k(ent_table, rel_table,
             h_list.astype(jnp.int32), t_list.astype(jnp.int32),
             r_list.astype(jnp.int32))
